# baseline (speedup 1.0000x reference)
import functools

import jax
import jax.numpy as jnp
from jax import lax
from jax.experimental import pallas as pl
from jax.experimental.pallas import tpu as pltpu
from jax.experimental.pallas import tpu_sc as plsc

N = 10000
E = 160000
ND = 256
ERD = 16
EAD = 16
ACD = ND + ERD + EAD

NC = 2
NS = 16
NW = NC * NS

_vmesh = plsc.VectorSubcoreMesh(core_axis_name="c", subcore_axis_name="s")


_EPW = E // NW
_ACH = 200
_ANCH = _EPW // _ACH


@functools.partial(
    pl.kernel,
    mesh=_vmesh,
    out_type=jax.ShapeDtypeStruct((E, ACD), jnp.float32),
    scratch_types=[
        pltpu.VMEM((_ACH,), jnp.int32),
        pltpu.VMEM((_ACH,), jnp.int32),
        pltpu.VMEM((_ACH, ACD), jnp.float32),
    ],
    compiler_params=pltpu.CompilerParams(use_tc_tiling_on_sc=False),
)
def _sc_gather_compose(r_hbm, src_hbm, x2_hbm, srcv, src2v, rows):
    wid = lax.axis_index("s") * NC + lax.axis_index("c")

    @pl.loop(0, _ANCH)
    def _(k):
        base = wid * _EPW + k * _ACH
        pltpu.sync_copy(src_hbm.at[pl.ds(base, _ACH)], srcv)
        pltpu.sync_copy(src_hbm.at[srcv], src2v)
        pltpu.sync_copy(r_hbm.at[src2v], rows)
        pltpu.sync_copy(rows, x2_hbm.at[pl.ds(base, _ACH)])


_HF = ACD // 2
_NRPS = 626
_NPAD = NS * _NRPS
_CCH = 200
_EPS = E // NS
_CNCH = _EPS // _CCH


@functools.partial(
    pl.kernel,
    mesh=_vmesh,
    out_type=[
        jax.ShapeDtypeStruct((_NPAD, _HF), jnp.float32),
        jax.ShapeDtypeStruct((_NPAD, _HF), jnp.float32),
    ],
    scratch_types=[
        pltpu.VMEM((_CCH,), jnp.int32),
        pltpu.VMEM((_CCH, _HF), jnp.float32),
        pltpu.VMEM_SHARED((_NPAD, _HF), jnp.float32),
    ],
    compiler_params=pltpu.CompilerParams(use_tc_tiling_on_sc=False),
)
def _sc_scatter_add(sub_hbm, dst_hbm, zeros_hbm, m0_hbm, m1_hbm,
                    dstv, rows, acc):
    cid = lax.axis_index("c")
    sid = lax.axis_index("s")

    pltpu.sync_copy(zeros_hbm, acc.at[pl.ds(sid * _NRPS, _NRPS)])
    plsc.subcore_barrier()

    @pl.loop(0, _CNCH)
    def _(k):
        base = sid * _EPS + k * _CCH
        pltpu.sync_copy(dst_hbm.at[pl.ds(base, _CCH)], dstv)

        @pl.when(cid == 0)
        def _():
            pltpu.sync_copy(sub_hbm.at[0, pl.ds(base, _CCH)], rows)

        @pl.when(cid == 1)
        def _():
            pltpu.sync_copy(sub_hbm.at[1, pl.ds(base, _CCH)], rows)

        pltpu.sync_copy(rows, acc.at[dstv], add=True)

    plsc.subcore_barrier()
    out_rows = pl.ds(sid * _NRPS, _NRPS)

    @pl.when(cid == 0)
    def _():
        pltpu.sync_copy(acc.at[out_rows], m0_hbm.at[out_rows])

    @pl.when(cid == 1)
    def _():
        pltpu.sync_copy(acc.at[out_rows], m1_hbm.at[out_rows])


_ECH = 200


@functools.partial(
    pl.kernel,
    mesh=_vmesh,
    out_type=[
        jax.ShapeDtypeStruct((E, ACD), jnp.float32),
        jax.ShapeDtypeStruct((E, ACD), jnp.float32),
    ],
    scratch_types=[
        pltpu.VMEM((_ECH,), jnp.int32),
        pltpu.VMEM((_ECH,), jnp.int32),
        pltpu.VMEM((_ECH, ACD), jnp.float32),
        pltpu.VMEM((_ECH, ACD), jnp.float32),
    ],
    compiler_params=pltpu.CompilerParams(use_tc_tiling_on_sc=False),
)
def _sc_gather_pair(p_hbm, q_hbm, src_hbm, dst_hbm, s_hbm, t_hbm,
                    srcv, dstv, rs, rt):
    wid = lax.axis_index("s") * NC + lax.axis_index("c")

    @pl.loop(0, _EPW // _ECH)
    def _(k):
        base = wid * _EPW + k * _ECH
        sl = pl.ds(base, _ECH)
        pltpu.sync_copy(src_hbm.at[sl], srcv)
        pltpu.sync_copy(dst_hbm.at[sl], dstv)
        pltpu.sync_copy(p_hbm.at[srcv], rs)
        pltpu.sync_copy(q_hbm.at[dstv], rt)
        pltpu.sync_copy(rs, s_hbm.at[sl])
        pltpu.sync_copy(rt, t_hbm.at[sl])


_BR = 2000


def _tc_pre_body(nf, wa, o):
    o[...] = jnp.dot(nf[...], wa[...], preferred_element_type=jnp.float32)


def _tc_pre(nf, wa):
    return pl.pallas_call(
        _tc_pre_body,
        grid=(N // _BR,),
        in_specs=[
            pl.BlockSpec((_BR, ND), lambda i: (i, 0)),
            pl.BlockSpec((ND, ACD), lambda i: (0, 0)),
        ],
        out_specs=pl.BlockSpec((_BR, ACD), lambda i: (i, 0)),
        out_shape=jax.ShapeDtypeStruct((N, ACD), jnp.float32),
    )(nf, wa)


_BE = 2000


def _tc_sub_body(x2, er, ea, wb, wc, b, o):
    acc = x2[...] + b[...]
    acc += jnp.dot(er[...], wb[...], preferred_element_type=jnp.float32)
    acc += jnp.dot(ea[...], wc[...], preferred_element_type=jnp.float32)
    s = jax.nn.sigmoid(acc)
    o[0] = s[:, :_HF]
    o[1] = s[:, _HF:]


def _tc_sub(x2, er, ea, wb, wc, b):
    return pl.pallas_call(
        _tc_sub_body,
        grid=(E // _BE,),
        in_specs=[
            pl.BlockSpec((_BE, ACD), lambda i: (i, 0)),
            pl.BlockSpec((_BE, ERD), lambda i: (i, 0)),
            pl.BlockSpec((_BE, EAD), lambda i: (i, 0)),
            pl.BlockSpec((ERD, ACD), lambda i: (0, 0)),
            pl.BlockSpec((EAD, ACD), lambda i: (0, 0)),
            pl.BlockSpec((1, ACD), lambda i: (0, 0)),
        ],
        out_specs=pl.BlockSpec((2, _BE, _HF), lambda i: (0, i, 0)),
        out_shape=jax.ShapeDtypeStruct((2, E, _HF), jnp.float32),
    )(x2, er, ea, wb, wc, b)


_BN = 1000


def _tc_node_body(m0, m1, nf, w0a, w0b, b0, w1, b1, w2, b2, w3, b3,
                  ws, bs, wn, bn, ewa, wa, onfu, op, oq):
    h = jnp.dot(m0[...], w0a[...], preferred_element_type=jnp.float32)
    h += jnp.dot(m1[...], w0b[...], preferred_element_type=jnp.float32)
    h = jax.nn.relu(h + b0[...])
    h = jax.nn.relu(jnp.dot(h, w1[...], preferred_element_type=jnp.float32)
                    + b1[...])
    h = jax.nn.relu(jnp.dot(h, w2[...], preferred_element_type=jnp.float32)
                    + b2[...])
    h = jnp.dot(h, w3[...], preferred_element_type=jnp.float32) + b3[...]
    nfv = nf[...]
    z = jnp.dot(nfv, ws[...], preferred_element_type=jnp.float32) + bs[...]
    z += jnp.dot(h, wn[...], preferred_element_type=jnp.float32) + bn[...]
    nfu = jax.nn.sigmoid(z) + nfv
    onfu[...] = nfu
    op[...] = jnp.dot(nfu, ewa[...], preferred_element_type=jnp.float32)
    oq[...] = jnp.dot(nfu, wa[...], preferred_element_type=jnp.float32)


def _tc_node(m0, m1, nf, w0a, w0b, b0, w1, b1, w2, b2, w3, b3, ws, bs,
             wn, bn, ewa, wa):
    full = lambda r, c: pl.BlockSpec((r, c), lambda i: (0, 0))
    return pl.pallas_call(
        _tc_node_body,
        grid=(N // _BN,),
        in_specs=[
            pl.BlockSpec((_BN, _HF), lambda i: (i, 0)),
            pl.BlockSpec((_BN, _HF), lambda i: (i, 0)),
            pl.BlockSpec((_BN, ND), lambda i: (i, 0)),
            full(_HF, 176), full(_HF, 176), full(1, 176),
            full(176, 64), full(1, 64),
            full(64, 128), full(1, 128),
            full(128, ND), full(1, ND),
            full(ND, ND), full(1, ND),
            full(ND, ND), full(1, ND),
            full(ND, ACD), full(ND, ACD),
        ],
        out_specs=[
            pl.BlockSpec((_BN, ND), lambda i: (i, 0)),
            pl.BlockSpec((_BN, ACD), lambda i: (i, 0)),
            pl.BlockSpec((_BN, ACD), lambda i: (i, 0)),
        ],
        out_shape=[
            jax.ShapeDtypeStruct((N, ND), jnp.float32),
            jax.ShapeDtypeStruct((N, ACD), jnp.float32),
            jax.ShapeDtypeStruct((N, ACD), jnp.float32),
        ],
    )(m0, m1, nf, w0a, w0b, b0, w1, b1, w2, b2, w3, b3, ws, bs, wn, bn,
      ewa, wa)


_BF = 2000


def _tc_edge_body(s, t, er, ea, ewb, ewc, eb, wb, wc, b,
                  v0, c0, v1, c1, v2, c2, v3, c3, o):
    a1 = s[...] + eb[...]
    a1 += jnp.dot(er[...], ewb[...], preferred_element_type=jnp.float32)
    a1 += jnp.dot(ea[...], ewc[...], preferred_element_type=jnp.float32)
    a2 = t[...] + b[...]
    a2 += jnp.dot(er[...], wb[...], preferred_element_type=jnp.float32)
    a2 += jnp.dot(ea[...], wc[...], preferred_element_type=jnp.float32)
    g = jax.nn.sigmoid(a1) + jax.nn.sigmoid(a2)
    g = jax.nn.relu(jnp.dot(g, v0[...], preferred_element_type=jnp.float32)
                    + c0[...])
    g = jax.nn.relu(jnp.dot(g, v1[...], preferred_element_type=jnp.float32)
                    + c1[...])
    g = jax.nn.relu(jnp.dot(g, v2[...], preferred_element_type=jnp.float32)
                    + c2[...])
    o[...] = jnp.dot(g, v3[...], preferred_element_type=jnp.float32) + c3[...]


def _tc_edge(s, t, er, ea, ewb, ewc, eb, wb, wc, b,
             v0, c0, v1, c1, v2, c2, v3, c3):
    full = lambda r, c: pl.BlockSpec((r, c), lambda i: (0, 0))
    return pl.pallas_call(
        _tc_edge_body,
        grid=(E // _BF,),
        in_specs=[
            pl.BlockSpec((_BF, ACD), lambda i: (i, 0)),
            pl.BlockSpec((_BF, ACD), lambda i: (i, 0)),
            pl.BlockSpec((_BF, ERD), lambda i: (i, 0)),
            pl.BlockSpec((_BF, EAD), lambda i: (i, 0)),
            full(ERD, ACD), full(EAD, ACD), full(1, ACD),
            full(ERD, ACD), full(EAD, ACD), full(1, ACD),
            full(ACD, 148), full(1, 148),
            full(148, 8), full(1, 8),
            full(8, 16), full(1, 16),
            full(16, 32), full(1, 32),
        ],
        out_specs=pl.BlockSpec((_BF, 32), lambda i: (i, 0)),
        out_shape=jax.ShapeDtypeStruct((E, 32), jnp.float32),
    )(s, t, er, ea, ewb, ewc, eb, wb, wc, b,
      v0, c0, v1, c1, v2, c2, v3, c3)


def kernel(node_features, edge_radial, edge_angular, edge_index,
           mlp1_w, mlp1_b, mlp2_w0, mlp2_b0, mlp2_w1, mlp2_b1,
           mlp2_w2, mlp2_b2, mlp2_w3, mlp2_b3, self_w, self_b,
           neigh_w, neigh_b, emlp1_w, emlp1_b, emlp2_w0, emlp2_b0,
           emlp2_w1, emlp2_b1, emlp2_w2, emlp2_b2, emlp2_w3, emlp2_b3):
    src = edge_index[0]
    dst = edge_index[1]

    w1t = mlp1_w.T
    wa, wb, wc = w1t[:ND], w1t[ND:ND + ERD], w1t[ND + ERD:]
    b1r = mlp1_b[None, :]
    e1t = emlp1_w.T
    ewa, ewb, ewc = e1t[:ND], e1t[ND:ND + ERD], e1t[ND + ERD:]
    eb1r = emlp1_b[None, :]

    w0t = mlp2_w0.T
    w0a, w0b = w0t[:_HF], w0t[_HF:]

    r = _tc_pre(node_features, wa)
    x2r = _sc_gather_compose(r, src)
    sub = _tc_sub(x2r, edge_radial, edge_angular, wb, wc, b1r)

    zeros = jnp.zeros((_NRPS, _HF), jnp.float32)
    m0, m1 = _sc_scatter_add(sub, dst, zeros)

    nfu, p, q = _tc_node(m0[:N], m1[:N], node_features,
                         w0a, w0b, mlp2_b0[None, :],
                         mlp2_w1.T, mlp2_b1[None, :],
                         mlp2_w2.T, mlp2_b2[None, :],
                         mlp2_w3.T, mlp2_b3[None, :],
                         self_w.T, self_b[None, :],
                         neigh_w.T, neigh_b[None, :],
                         ewa, wa)

    s_rows, t_rows = _sc_gather_pair(p, q, src, dst)

    e = _tc_edge(s_rows, t_rows, edge_radial, edge_angular,
                 ewb, ewc, eb1r, wb, wc, b1r,
                 emlp2_w0.T, emlp2_b0[None, :],
                 emlp2_w1.T, emlp2_b1[None, :],
                 emlp2_w2.T, emlp2_b2[None, :],
                 emlp2_w3.T, emlp2_b3[None, :])

    return (nfu, e)


# speedup vs baseline: 1.6236x; 1.6236x over previous
import functools

import jax
import jax.numpy as jnp
from jax import lax
from jax.experimental import pallas as pl
from jax.experimental.pallas import tpu as pltpu
from jax.experimental.pallas import tpu_sc as plsc

N = 10000
E = 160000
ND = 256
ERD = 16
EAD = 16
ACD = ND + ERD + EAD
HD0 = 176
ZP = 176

NC = 2
NS = 16
NW = NC * NS

_vmesh = plsc.VectorSubcoreMesh(core_axis_name="c", subcore_axis_name="s")


_EPW = E // NW
_ACH = 200
_ANCH = _EPW // _ACH


@functools.partial(
    pl.kernel,
    mesh=_vmesh,
    out_type=jax.ShapeDtypeStruct((E, ND), jnp.float32),
    scratch_types=[
        pltpu.VMEM((_ACH,), jnp.int32),
        pltpu.VMEM((_ACH,), jnp.int32),
        pltpu.VMEM((_ACH, ND), jnp.float32),
    ],
)
def _sc_gather_compose(nf_hbm, src_hbm, x2_hbm, srcv, src2v, rows):
    wid = lax.axis_index("s") * NC + lax.axis_index("c")

    @pl.loop(0, _ANCH)
    def _(k):
        base = wid * _EPW + k * _ACH
        pltpu.sync_copy(src_hbm.at[pl.ds(base, _ACH)], srcv)
        pltpu.sync_copy(src_hbm.at[srcv], src2v)
        pltpu.sync_copy(nf_hbm.at[src2v], rows)
        pltpu.sync_copy(rows, x2_hbm.at[pl.ds(base, _ACH)])


_NRPS = 626
_NPAD = NS * _NRPS
_CCH = 100
_EPS = E // NC // NS
_CNCH = _EPS // _CCH


@functools.partial(
    pl.kernel,
    mesh=_vmesh,
    out_type=[
        jax.ShapeDtypeStruct((_NPAD, HD0), jnp.float32),
        jax.ShapeDtypeStruct((_NPAD, HD0), jnp.float32),
    ],
    scratch_types=[
        pltpu.VMEM((1, _CCH), jnp.int32),
        pltpu.VMEM((_CCH, HD0), jnp.float32),
        pltpu.VMEM_SHARED((_NPAD, HD0), jnp.float32),
    ],
    compiler_params=pltpu.CompilerParams(use_tc_tiling_on_sc=False),
)
def _sc_scatter_add(z_hbm, dst_hbm, zeros_hbm, m0_hbm, m1_hbm,
                    dstv, rows, acc):
    cid = lax.axis_index("c")
    sid = lax.axis_index("s")

    pltpu.sync_copy(zeros_hbm, acc.at[pl.ds(sid * _NRPS, _NRPS)])
    plsc.subcore_barrier()

    @pl.loop(0, _CNCH)
    def _(k):
        chunk = (cid * (E // NC) + sid * _EPS) // _CCH + k
        pltpu.sync_copy(dst_hbm.at[pl.ds(chunk, 1)], dstv)
        pltpu.sync_copy(z_hbm.at[pl.ds(chunk * _CCH, _CCH)], rows)
        pltpu.sync_copy(rows, acc.at[dstv.at[0]], add=True)

    plsc.subcore_barrier()
    out_rows = pl.ds(sid * _NRPS, _NRPS)

    @pl.when(cid == 0)
    def _():
        pltpu.sync_copy(acc.at[out_rows], m0_hbm.at[out_rows])

    @pl.when(cid == 1)
    def _():
        pltpu.sync_copy(acc.at[out_rows], m1_hbm.at[out_rows])


@functools.partial(
    pl.kernel,
    mesh=_vmesh,
    out_type=[
        jax.ShapeDtypeStruct((E, ND), jnp.float32),
        jax.ShapeDtypeStruct((E, ND), jnp.float32),
    ],
    scratch_types=[
        pltpu.VMEM((_ACH,), jnp.int32),
        pltpu.VMEM((_ACH,), jnp.int32),
        pltpu.VMEM((_ACH, ND), jnp.float32),
        pltpu.VMEM((_ACH, ND), jnp.float32),
    ],
)
def _sc_gather_pair(nfu_hbm, src_hbm, dst_hbm, s_hbm, t_hbm,
                    srcv, dstv, rs, rt):
    wid = lax.axis_index("s") * NC + lax.axis_index("c")

    @pl.loop(0, _ANCH)
    def _(k):
        base = wid * _EPW + k * _ACH
        sl = pl.ds(base, _ACH)
        pltpu.sync_copy(src_hbm.at[sl], srcv)
        pltpu.sync_copy(dst_hbm.at[sl], dstv)
        pltpu.sync_copy(nfu_hbm.at[srcv], rs)
        pltpu.sync_copy(nfu_hbm.at[dstv], rt)
        pltpu.sync_copy(rs, s_hbm.at[sl])
        pltpu.sync_copy(rt, t_hbm.at[sl])


_BE = 2000


def _tc_sub_body(x2, er, ea, wa, wb, wc, b, w0p, o):
    acc = jnp.dot(x2[...], wa[...], preferred_element_type=jnp.float32)
    acc += jnp.dot(er[...], wb[...], preferred_element_type=jnp.float32)
    acc += jnp.dot(ea[...], wc[...], preferred_element_type=jnp.float32)
    sub = jax.nn.sigmoid(acc + b[...])
    o[...] = jnp.dot(sub, w0p[...], preferred_element_type=jnp.float32)


def _tc_sub(x2, er, ea, wa, wb, wc, b, w0p):
    return pl.pallas_call(
        _tc_sub_body,
        grid=(E // _BE,),
        in_specs=[
            pl.BlockSpec((_BE, ND), lambda i: (i, 0)),
            pl.BlockSpec((_BE, ERD), lambda i: (i, 0)),
            pl.BlockSpec((_BE, EAD), lambda i: (i, 0)),
            pl.BlockSpec((ND, ACD), lambda i: (0, 0)),
            pl.BlockSpec((ERD, ACD), lambda i: (0, 0)),
            pl.BlockSpec((EAD, ACD), lambda i: (0, 0)),
            pl.BlockSpec((1, ACD), lambda i: (0, 0)),
            pl.BlockSpec((ACD, ZP), lambda i: (0, 0)),
        ],
        out_specs=pl.BlockSpec((_BE, ZP), lambda i: (i, 0)),
        out_shape=jax.ShapeDtypeStruct((E, ZP), jnp.float32),
    )(x2, er, ea, wa, wb, wc, b, w0p)


_BN = 1000


def _tc_node_body(m0, m1, nf, b0, w1, b1, w2, b2, w3, b3,
                  ws, bs, wn, bn, o):
    h = jax.nn.relu(m0[...] + m1[...] + b0[...])
    h = jax.nn.relu(jnp.dot(h, w1[...], preferred_element_type=jnp.float32)
                    + b1[...])
    h = jax.nn.relu(jnp.dot(h, w2[...], preferred_element_type=jnp.float32)
                    + b2[...])
    h = jnp.dot(h, w3[...], preferred_element_type=jnp.float32) + b3[...]
    nfv = nf[...]
    z = jnp.dot(nfv, ws[...], preferred_element_type=jnp.float32) + bs[...]
    z += jnp.dot(h, wn[...], preferred_element_type=jnp.float32) + bn[...]
    o[...] = jax.nn.sigmoid(z) + nfv


def _tc_node(m0, m1, nf, b0, w1, b1, w2, b2, w3, b3, ws, bs, wn, bn):
    full = lambda r, c: pl.BlockSpec((r, c), lambda i: (0, 0))
    return pl.pallas_call(
        _tc_node_body,
        grid=(N // _BN,),
        in_specs=[
            pl.BlockSpec((_BN, HD0), lambda i: (i, 0)),
            pl.BlockSpec((_BN, HD0), lambda i: (i, 0)),
            pl.BlockSpec((_BN, ND), lambda i: (i, 0)),
            full(1, HD0),
            full(HD0, 64), full(1, 64),
            full(64, 128), full(1, 128),
            full(128, ND), full(1, ND),
            full(ND, ND), full(1, ND),
            full(ND, ND), full(1, ND),
        ],
        out_specs=pl.BlockSpec((_BN, ND), lambda i: (i, 0)),
        out_shape=jax.ShapeDtypeStruct((N, ND), jnp.float32),
    )(m0, m1, nf, b0, w1, b1, w2, b2, w3, b3, ws, bs, wn, bn)


_BF = 2000


def _tc_edge_body(s, t, er, ea, ewa, ewb, ewc, eb, wa, wb, wc, b,
                  v0, c0, v1, c1, v2, c2, v3, c3, o):
    a1 = jnp.dot(s[...], ewa[...], preferred_element_type=jnp.float32)
    a1 += jnp.dot(er[...], ewb[...], preferred_element_type=jnp.float32)
    a1 += jnp.dot(ea[...], ewc[...], preferred_element_type=jnp.float32)
    a2 = jnp.dot(t[...], wa[...], preferred_element_type=jnp.float32)
    a2 += jnp.dot(er[...], wb[...], preferred_element_type=jnp.float32)
    a2 += jnp.dot(ea[...], wc[...], preferred_element_type=jnp.float32)
    g = jax.nn.sigmoid(a1 + eb[...]) + jax.nn.sigmoid(a2 + b[...])
    g = jax.nn.relu(jnp.dot(g, v0[...], preferred_element_type=jnp.float32)
                    + c0[...])
    g = jax.nn.relu(jnp.dot(g, v1[...], preferred_element_type=jnp.float32)
                    + c1[...])
    g = jax.nn.relu(jnp.dot(g, v2[...], preferred_element_type=jnp.float32)
                    + c2[...])
    o[...] = jnp.dot(g, v3[...], preferred_element_type=jnp.float32) + c3[...]


def _tc_edge(s, t, er, ea, ewa, ewb, ewc, eb, wa, wb, wc, b,
             v0, c0, v1, c1, v2, c2, v3, c3):
    full = lambda r, c: pl.BlockSpec((r, c), lambda i: (0, 0))
    return pl.pallas_call(
        _tc_edge_body,
        grid=(E // _BF,),
        in_specs=[
            pl.BlockSpec((_BF, ND), lambda i: (i, 0)),
            pl.BlockSpec((_BF, ND), lambda i: (i, 0)),
            pl.BlockSpec((_BF, ERD), lambda i: (i, 0)),
            pl.BlockSpec((_BF, EAD), lambda i: (i, 0)),
            full(ND, ACD), full(ERD, ACD), full(EAD, ACD), full(1, ACD),
            full(ND, ACD), full(ERD, ACD), full(EAD, ACD), full(1, ACD),
            full(ACD, 148), full(1, 148),
            full(148, 8), full(1, 8),
            full(8, 16), full(1, 16),
            full(16, 32), full(1, 32),
        ],
        out_specs=pl.BlockSpec((_BF, 32), lambda i: (i, 0)),
        out_shape=jax.ShapeDtypeStruct((E, 32), jnp.float32),
    )(s, t, er, ea, ewa, ewb, ewc, eb, wa, wb, wc, b,
      v0, c0, v1, c1, v2, c2, v3, c3)


def kernel(node_features, edge_radial, edge_angular, edge_index,
           mlp1_w, mlp1_b, mlp2_w0, mlp2_b0, mlp2_w1, mlp2_b1,
           mlp2_w2, mlp2_b2, mlp2_w3, mlp2_b3, self_w, self_b,
           neigh_w, neigh_b, emlp1_w, emlp1_b, emlp2_w0, emlp2_b0,
           emlp2_w1, emlp2_b1, emlp2_w2, emlp2_b2, emlp2_w3, emlp2_b3):
    src = edge_index[0]
    dst = edge_index[1]

    w1t = mlp1_w.T
    wa, wb, wc = w1t[:ND], w1t[ND:ND + ERD], w1t[ND + ERD:]
    b1r = mlp1_b[None, :]
    e1t = emlp1_w.T
    ewa, ewb, ewc = e1t[:ND], e1t[ND:ND + ERD], e1t[ND + ERD:]
    eb1r = emlp1_b[None, :]

    w0p = mlp2_w0.T

    x2 = _sc_gather_compose(node_features, src)
    z = _tc_sub(x2, edge_radial, edge_angular, wa, wb, wc, b1r, w0p)

    zeros = jnp.zeros((_NRPS, HD0), jnp.float32)
    m0, m1 = _sc_scatter_add(z, dst.reshape(E // _CCH, _CCH), zeros)

    nfu = _tc_node(m0[:N], m1[:N], node_features,
                   mlp2_b0[None, :],
                   mlp2_w1.T, mlp2_b1[None, :],
                   mlp2_w2.T, mlp2_b2[None, :],
                   mlp2_w3.T, mlp2_b3[None, :],
                   self_w.T, self_b[None, :],
                   neigh_w.T, neigh_b[None, :])

    s_rows, t_rows = _sc_gather_pair(nfu, src, dst)

    e = _tc_edge(s_rows, t_rows, edge_radial, edge_angular,
                 ewa, ewb, ewc, eb1r, wa, wb, wc, b1r,
                 emlp2_w0.T, emlp2_b0[None, :],
                 emlp2_w1.T, emlp2_b1[None, :],
                 emlp2_w2.T, emlp2_b2[None, :],
                 emlp2_w3.T, emlp2_b3[None, :])

    return (nfu, e)


# speedup vs baseline: 1.6391x; 1.0095x over previous
import functools

import jax
import jax.numpy as jnp
from jax import lax
from jax.experimental import pallas as pl
from jax.experimental.pallas import tpu as pltpu
from jax.experimental.pallas import tpu_sc as plsc

N = 10000
E = 160000
ND = 256
ERD = 16
EAD = 16
ACD = ND + ERD + EAD
HD0 = 176

NC = 2
NS = 16
NW = NC * NS
_EPW = E // NW

_H_EPW = (2600, 2400)
_H_CH = (104, 96)

_NRPS = 626
_NPAD = NS * _NRPS

_vmesh = plsc.VectorSubcoreMesh(core_axis_name="c", subcore_axis_name="s")


def _mk_gather_compose(epw, ch):
    total = NW * epw
    nch = epw // ch

    @functools.partial(
        pl.kernel,
        mesh=_vmesh,
        out_type=jax.ShapeDtypeStruct((total, ND), jnp.float32),
        scratch_types=[
            pltpu.VMEM((1, ch), jnp.int32),
            pltpu.VMEM((1, ch), jnp.int32),
            pltpu.VMEM((ch, ND), jnp.float32),
        ],
    )
    def _gather_compose(nf_hbm, src2d_hbm, srcfull_hbm, x2_hbm,
                        srcv, src2v, rows):
        wid = lax.axis_index("s") * NC + lax.axis_index("c")

        @pl.loop(0, nch)
        def _(k):
            pltpu.sync_copy(src2d_hbm.at[pl.ds(wid * nch + k, 1)], srcv)
            pltpu.sync_copy(srcfull_hbm.at[srcv.at[0]], src2v.at[0])
            pltpu.sync_copy(nf_hbm.at[src2v.at[0]], rows)
            pltpu.sync_copy(rows, x2_hbm.at[pl.ds(wid * epw + k * ch, ch)])

    return _gather_compose


def _mk_scatter_add(epw, ch):
    nch = epw // ch

    @functools.partial(
        pl.kernel,
        mesh=_vmesh,
        out_type=[
            jax.ShapeDtypeStruct((_NPAD, HD0), jnp.float32),
            jax.ShapeDtypeStruct((_NPAD, HD0), jnp.float32),
        ],
        scratch_types=[
            pltpu.VMEM((1, ch), jnp.int32),
            pltpu.VMEM((ch, HD0), jnp.float32),
            pltpu.VMEM_SHARED((_NPAD, HD0), jnp.float32),
        ],
        compiler_params=pltpu.CompilerParams(use_tc_tiling_on_sc=False),
    )
    def _scatter_add(z_hbm, dst2d_hbm, zeros_hbm, m0_hbm, m1_hbm,
                     dstv, rows, acc):
        cid = lax.axis_index("c")
        sid = lax.axis_index("s")

        pltpu.sync_copy(zeros_hbm, acc.at[pl.ds(sid * _NRPS, _NRPS)])
        plsc.subcore_barrier()

        @pl.loop(0, nch)
        def _(k):
            crow = (cid * NS + sid) * nch + k
            pltpu.sync_copy(dst2d_hbm.at[pl.ds(crow, 1)], dstv)
            pltpu.sync_copy(z_hbm.at[pl.ds(crow * ch, ch)], rows)
            pltpu.sync_copy(rows, acc.at[dstv.at[0]], add=True)

        plsc.subcore_barrier()
        out_rows = pl.ds(sid * _NRPS, _NRPS)

        @pl.when(cid == 0)
        def _():
            pltpu.sync_copy(acc.at[out_rows], m0_hbm.at[out_rows])

        @pl.when(cid == 1)
        def _():
            pltpu.sync_copy(acc.at[out_rows], m1_hbm.at[out_rows])

    return _scatter_add


def _mk_gather_pair(epw, ch):
    total = NW * epw
    nch = epw // ch

    @functools.partial(
        pl.kernel,
        mesh=_vmesh,
        out_type=[
            jax.ShapeDtypeStruct((total, ND), jnp.float32),
            jax.ShapeDtypeStruct((total, ND), jnp.float32),
        ],
        scratch_types=[
            pltpu.VMEM((1, ch), jnp.int32),
            pltpu.VMEM((1, ch), jnp.int32),
            pltpu.VMEM((ch, ND), jnp.float32),
            pltpu.VMEM((ch, ND), jnp.float32),
        ],
    )
    def _gather_pair(nfu_hbm, src2d_hbm, dst2d_hbm, s_hbm, t_hbm,
                     srcv, dstv, rs, rt):
        wid = lax.axis_index("s") * NC + lax.axis_index("c")

        @pl.loop(0, nch)
        def _(k):
            crow = pl.ds(wid * nch + k, 1)
            pltpu.sync_copy(src2d_hbm.at[crow], srcv)
            pltpu.sync_copy(dst2d_hbm.at[crow], dstv)
            pltpu.sync_copy(nfu_hbm.at[srcv.at[0]], rs)
            pltpu.sync_copy(nfu_hbm.at[dstv.at[0]], rt)
            sl = pl.ds(wid * epw + k * ch, ch)
            pltpu.sync_copy(rs, s_hbm.at[sl])
            pltpu.sync_copy(rt, t_hbm.at[sl])

    return _gather_pair


_sc_a = tuple(_mk_gather_compose(_H_EPW[h], _H_CH[h]) for h in range(2))
_sc_c = tuple(_mk_scatter_add(_H_EPW[h], _H_CH[h]) for h in range(2))
_sc_e = tuple(_mk_gather_pair(_H_EPW[h], _H_CH[h]) for h in range(2))


def _tc_sub_body(x2, er, ea, wa, wb, wc, b, w0p, o):
    acc = jnp.dot(x2[...], wa[...], preferred_element_type=jnp.float32)
    acc += jnp.dot(er[...], wb[...], preferred_element_type=jnp.float32)
    acc += jnp.dot(ea[...], wc[...], preferred_element_type=jnp.float32)
    sub = jax.nn.sigmoid(acc + b[...])
    o[...] = jnp.dot(sub, w0p[...], preferred_element_type=jnp.float32)


def _tc_sub(x2, er, ea, wa, wb, wc, b, w0p):
    total = x2.shape[0]
    be = total // 32
    return pl.pallas_call(
        _tc_sub_body,
        grid=(32,),
        in_specs=[
            pl.BlockSpec((be, ND), lambda i: (i, 0)),
            pl.BlockSpec((be, ERD), lambda i: (i, 0)),
            pl.BlockSpec((be, EAD), lambda i: (i, 0)),
            pl.BlockSpec((ND, ACD), lambda i: (0, 0)),
            pl.BlockSpec((ERD, ACD), lambda i: (0, 0)),
            pl.BlockSpec((EAD, ACD), lambda i: (0, 0)),
            pl.BlockSpec((1, ACD), lambda i: (0, 0)),
            pl.BlockSpec((ACD, HD0), lambda i: (0, 0)),
        ],
        out_specs=pl.BlockSpec((be, HD0), lambda i: (i, 0)),
        out_shape=jax.ShapeDtypeStruct((total, HD0), jnp.float32),
    )(x2, er, ea, wa, wb, wc, b, w0p)


_BN = 1000


def _tc_node_body(m0a, m1a, m0b, m1b, nf, b0, w1, b1, w2, b2, w3, b3,
                  ws, bs, wn, bn, o):
    hpre = m0a[...] + m1a[...] + m0b[...] + m1b[...]
    h = jax.nn.relu(hpre + b0[...])
    h = jax.nn.relu(jnp.dot(h, w1[...], preferred_element_type=jnp.float32)
                    + b1[...])
    h = jax.nn.relu(jnp.dot(h, w2[...], preferred_element_type=jnp.float32)
                    + b2[...])
    h = jnp.dot(h, w3[...], preferred_element_type=jnp.float32) + b3[...]
    nfv = nf[...]
    z = jnp.dot(nfv, ws[...], preferred_element_type=jnp.float32) + bs[...]
    z += jnp.dot(h, wn[...], preferred_element_type=jnp.float32) + bn[...]
    o[...] = jax.nn.sigmoid(z) + nfv


def _tc_node(m0a, m1a, m0b, m1b, nf, b0, w1, b1, w2, b2, w3, b3,
             ws, bs, wn, bn):
    full = lambda r, c: pl.BlockSpec((r, c), lambda i: (0, 0))
    mspec = pl.BlockSpec((_BN, HD0), lambda i: (i, 0))
    return pl.pallas_call(
        _tc_node_body,
        grid=(N // _BN,),
        in_specs=[
            mspec, mspec, mspec, mspec,
            pl.BlockSpec((_BN, ND), lambda i: (i, 0)),
            full(1, HD0),
            full(HD0, 64), full(1, 64),
            full(64, 128), full(1, 128),
            full(128, ND), full(1, ND),
            full(ND, ND), full(1, ND),
            full(ND, ND), full(1, ND),
        ],
        out_specs=pl.BlockSpec((_BN, ND), lambda i: (i, 0)),
        out_shape=jax.ShapeDtypeStruct((N, ND), jnp.float32),
    )(m0a, m1a, m0b, m1b, nf, b0, w1, b1, w2, b2, w3, b3, ws, bs, wn, bn)


def _tc_edge_body(s, t, er, ea, ewa, ewb, ewc, eb, wa, wb, wc, b,
                  v0, c0, v1, c1, v2, c2, v3, c3, o):
    a1 = jnp.dot(s[...], ewa[...], preferred_element_type=jnp.float32)
    a1 += jnp.dot(er[...], ewb[...], preferred_element_type=jnp.float32)
    a1 += jnp.dot(ea[...], ewc[...], preferred_element_type=jnp.float32)
    a2 = jnp.dot(t[...], wa[...], preferred_element_type=jnp.float32)
    a2 += jnp.dot(er[...], wb[...], preferred_element_type=jnp.float32)
    a2 += jnp.dot(ea[...], wc[...], preferred_element_type=jnp.float32)
    g = jax.nn.sigmoid(a1 + eb[...]) + jax.nn.sigmoid(a2 + b[...])
    g = jax.nn.relu(jnp.dot(g, v0[...], preferred_element_type=jnp.float32)
                    + c0[...])
    g = jax.nn.relu(jnp.dot(g, v1[...], preferred_element_type=jnp.float32)
                    + c1[...])
    g = jax.nn.relu(jnp.dot(g, v2[...], preferred_element_type=jnp.float32)
                    + c2[...])
    o[...] = jnp.dot(g, v3[...], preferred_element_type=jnp.float32) + c3[...]


def _tc_edge(s, t, er, ea, ewa, ewb, ewc, eb, wa, wb, wc, b,
             v0, c0, v1, c1, v2, c2, v3, c3):
    total = s.shape[0]
    bf = total // 32
    full = lambda r, c: pl.BlockSpec((r, c), lambda i: (0, 0))
    return pl.pallas_call(
        _tc_edge_body,
        grid=(32,),
        in_specs=[
            pl.BlockSpec((bf, ND), lambda i: (i, 0)),
            pl.BlockSpec((bf, ND), lambda i: (i, 0)),
            pl.BlockSpec((bf, ERD), lambda i: (i, 0)),
            pl.BlockSpec((bf, EAD), lambda i: (i, 0)),
            full(ND, ACD), full(ERD, ACD), full(EAD, ACD), full(1, ACD),
            full(ND, ACD), full(ERD, ACD), full(EAD, ACD), full(1, ACD),
            full(ACD, 148), full(1, 148),
            full(148, 8), full(1, 8),
            full(8, 16), full(1, 16),
            full(16, 32), full(1, 32),
        ],
        out_specs=pl.BlockSpec((bf, 32), lambda i: (i, 0)),
        out_shape=jax.ShapeDtypeStruct((total, 32), jnp.float32),
    )(s, t, er, ea, ewa, ewb, ewc, eb, wa, wb, wc, b,
      v0, c0, v1, c1, v2, c2, v3, c3)


def kernel(node_features, edge_radial, edge_angular, edge_index,
           mlp1_w, mlp1_b, mlp2_w0, mlp2_b0, mlp2_w1, mlp2_b1,
           mlp2_w2, mlp2_b2, mlp2_w3, mlp2_b3, self_w, self_b,
           neigh_w, neigh_b, emlp1_w, emlp1_b, emlp2_w0, emlp2_b0,
           emlp2_w1, emlp2_b1, emlp2_w2, emlp2_b2, emlp2_w3, emlp2_b3):
    src = edge_index[0]
    dst = edge_index[1]

    w1t = mlp1_w.T
    wa, wb, wc = w1t[:ND], w1t[ND:ND + ERD], w1t[ND + ERD:]
    b1r = mlp1_b[None, :]
    e1t = emlp1_w.T
    ewa, ewb, ewc = e1t[:ND], e1t[ND:ND + ERD], e1t[ND + ERD:]
    eb1r = emlp1_b[None, :]
    w0p = mlp2_w0.T
    zeros = jnp.zeros((_NRPS, HD0), jnp.float32)

    def _split(x):
        xw = x.reshape(NW, _EPW, *x.shape[1:])
        return (xw[:, :_H_EPW[0]].reshape(-1, *x.shape[1:]),
                xw[:, _H_EPW[0]:].reshape(-1, *x.shape[1:]))

    src_h = _split(src)
    dst_h = _split(dst)
    er_h = _split(edge_radial)
    ea_h = _split(edge_angular)
    src2d = tuple(src_h[h].reshape(-1, _H_CH[h]) for h in range(2))
    dst2d = tuple(dst_h[h].reshape(-1, _H_CH[h]) for h in range(2))

    z = [None, None]
    for h in range(2):
        x2 = _sc_a[h](node_features, src2d[h], src)
        z[h] = _tc_sub(x2, er_h[h], ea_h[h], wa, wb, wc, b1r, w0p)

    m0a, m1a = _sc_c[0](z[0], dst2d[0], zeros)
    m0b, m1b = _sc_c[1](z[1], dst2d[1], zeros)

    nfu = _tc_node(m0a[:N], m1a[:N], m0b[:N], m1b[:N], node_features,
                   mlp2_b0[None, :],
                   mlp2_w1.T, mlp2_b1[None, :],
                   mlp2_w2.T, mlp2_b2[None, :],
                   mlp2_w3.T, mlp2_b3[None, :],
                   self_w.T, self_b[None, :],
                   neigh_w.T, neigh_b[None, :])

    e_h = [None, None]
    for h in range(2):
        s_rows, t_rows = _sc_e[h](nfu, src2d[h], dst2d[h])
        e_h[h] = _tc_edge(s_rows, t_rows, er_h[h], ea_h[h],
                          ewa, ewb, ewc, eb1r, wa, wb, wc, b1r,
                          emlp2_w0.T, emlp2_b0[None, :],
                          emlp2_w1.T, emlp2_b1[None, :],
                          emlp2_w2.T, emlp2_b2[None, :],
                          emlp2_w3.T, emlp2_b3[None, :])

    e = jnp.concatenate([e_h[0].reshape(NW, _H_EPW[0], 32),
                         e_h[1].reshape(NW, _H_EPW[1], 32)],
                        axis=1).reshape(E, 32)
    return (nfu, e)


# speedup vs baseline: 1.6924x; 1.0325x over previous
import functools

import jax
import jax.numpy as jnp
from jax import lax
from jax.experimental import pallas as pl
from jax.experimental.pallas import tpu as pltpu
from jax.experimental.pallas import tpu_sc as plsc

N = 10000
E = 160000
ND = 256
ERD = 16
EAD = 16
ACD = ND + ERD + EAD
HD0 = 176

NC = 2
NS = 16
NW = NC * NS
_EPW = E // NW

_H_EPW = (2600, 2400)
_GCH = 200
_H_CH = (104, 96)

_NRPS = 626
_NPAD = NS * _NRPS

_vmesh = plsc.VectorSubcoreMesh(core_axis_name="c", subcore_axis_name="s")


def _mk_gather_compose(epw, ch):
    total = NW * epw
    nch = epw // ch

    @functools.partial(
        pl.kernel,
        mesh=_vmesh,
        out_type=jax.ShapeDtypeStruct((total, ND), jnp.float32),
        scratch_types=[
            pltpu.VMEM((1, ch), jnp.int32),
            pltpu.VMEM((1, ch), jnp.int32),
            pltpu.VMEM((ch, ND), jnp.float32),
        ],
    )
    def _gather_compose(nf_hbm, src2d_hbm, srcfull_hbm, x2_hbm,
                        srcv, src2v, rows):
        wid = lax.axis_index("s") * NC + lax.axis_index("c")

        @pl.loop(0, nch)
        def _(k):
            pltpu.sync_copy(src2d_hbm.at[pl.ds(wid * nch + k, 1)], srcv)
            pltpu.sync_copy(srcfull_hbm.at[srcv.at[0]], src2v.at[0])
            pltpu.sync_copy(nf_hbm.at[src2v.at[0]], rows)
            pltpu.sync_copy(rows, x2_hbm.at[pl.ds(wid * epw + k * ch, ch)])

    return _gather_compose


def _mk_scatter_add(epw, ch):
    nch = epw // ch

    @functools.partial(
        pl.kernel,
        mesh=_vmesh,
        out_type=[
            jax.ShapeDtypeStruct((_NPAD, HD0), jnp.float32),
            jax.ShapeDtypeStruct((_NPAD, HD0), jnp.float32),
        ],
        scratch_types=[
            pltpu.VMEM((1, ch), jnp.int32),
            pltpu.VMEM((ch, HD0), jnp.float32),
            pltpu.VMEM_SHARED((_NPAD, HD0), jnp.float32),
        ],
        compiler_params=pltpu.CompilerParams(use_tc_tiling_on_sc=False),
    )
    def _scatter_add(z_hbm, dst2d_hbm, zeros_hbm, m0_hbm, m1_hbm,
                     dstv, rows, acc):
        cid = lax.axis_index("c")
        sid = lax.axis_index("s")

        pltpu.sync_copy(zeros_hbm, acc.at[pl.ds(sid * _NRPS, _NRPS)])
        plsc.subcore_barrier()

        @pl.loop(0, nch)
        def _(k):
            crow = (cid * NS + sid) * nch + k
            pltpu.sync_copy(dst2d_hbm.at[pl.ds(crow, 1)], dstv)
            pltpu.sync_copy(z_hbm.at[pl.ds(crow * ch, ch)], rows)
            pltpu.sync_copy(rows, acc.at[dstv.at[0]], add=True)

        plsc.subcore_barrier()
        out_rows = pl.ds(sid * _NRPS, _NRPS)

        @pl.when(cid == 0)
        def _():
            pltpu.sync_copy(acc.at[out_rows], m0_hbm.at[out_rows])

        @pl.when(cid == 1)
        def _():
            pltpu.sync_copy(acc.at[out_rows], m1_hbm.at[out_rows])

    return _scatter_add


def _mk_gather_pair(epw, ch):
    total = NW * epw
    nch = epw // ch

    @functools.partial(
        pl.kernel,
        mesh=_vmesh,
        out_type=[
            jax.ShapeDtypeStruct((total, ND), jnp.float32),
            jax.ShapeDtypeStruct((total, ND), jnp.float32),
        ],
        scratch_types=[
            pltpu.VMEM((1, ch), jnp.int32),
            pltpu.VMEM((1, ch), jnp.int32),
            pltpu.VMEM((ch, ND), jnp.float32),
            pltpu.VMEM((ch, ND), jnp.float32),
        ],
    )
    def _gather_pair(nfu_hbm, src2d_hbm, dst2d_hbm, s_hbm, t_hbm,
                     srcv, dstv, rs, rt):
        wid = lax.axis_index("s") * NC + lax.axis_index("c")

        @pl.loop(0, nch)
        def _(k):
            crow = pl.ds(wid * nch + k, 1)
            pltpu.sync_copy(src2d_hbm.at[crow], srcv)
            pltpu.sync_copy(dst2d_hbm.at[crow], dstv)
            pltpu.sync_copy(nfu_hbm.at[srcv.at[0]], rs)
            pltpu.sync_copy(nfu_hbm.at[dstv.at[0]], rt)
            sl = pl.ds(wid * epw + k * ch, ch)
            pltpu.sync_copy(rs, s_hbm.at[sl])
            pltpu.sync_copy(rt, t_hbm.at[sl])

    return _gather_pair


_sc_a = tuple(_mk_gather_compose(_H_EPW[h], _GCH) for h in range(2))
_sc_c = tuple(_mk_scatter_add(_H_EPW[h], _H_CH[h]) for h in range(2))
_sc_e = tuple(_mk_gather_pair(_H_EPW[h], _GCH) for h in range(2))


def _tc_sub_body(x2, er, ea, wa, wb, wc, b, w0p, o):
    acc = jnp.dot(x2[...], wa[...], preferred_element_type=jnp.float32)
    acc += jnp.dot(er[...], wb[...], preferred_element_type=jnp.float32)
    acc += jnp.dot(ea[...], wc[...], preferred_element_type=jnp.float32)
    sub = jax.nn.sigmoid(acc + b[...])
    o[...] = jnp.dot(sub, w0p[...], preferred_element_type=jnp.float32)


def _tc_sub(x2, er, ea, wa, wb, wc, b, w0p):
    total = x2.shape[0]
    be = total // 32
    return pl.pallas_call(
        _tc_sub_body,
        grid=(32,),
        in_specs=[
            pl.BlockSpec((be, ND), lambda i: (i, 0)),
            pl.BlockSpec((be, ERD), lambda i: (i, 0)),
            pl.BlockSpec((be, EAD), lambda i: (i, 0)),
            pl.BlockSpec((ND, ACD), lambda i: (0, 0)),
            pl.BlockSpec((ERD, ACD), lambda i: (0, 0)),
            pl.BlockSpec((EAD, ACD), lambda i: (0, 0)),
            pl.BlockSpec((1, ACD), lambda i: (0, 0)),
            pl.BlockSpec((ACD, HD0), lambda i: (0, 0)),
        ],
        out_specs=pl.BlockSpec((be, HD0), lambda i: (i, 0)),
        out_shape=jax.ShapeDtypeStruct((total, HD0), jnp.float32),
    )(x2, er, ea, wa, wb, wc, b, w0p)


_BN = 1000


def _tc_node_body(m0a, m1a, m0b, m1b, nf, b0, w1, b1, w2, b2, w3, b3,
                  ws, bs, wn, bn, o):
    hpre = m0a[...] + m1a[...] + m0b[...] + m1b[...]
    h = jax.nn.relu(hpre + b0[...])
    h = jax.nn.relu(jnp.dot(h, w1[...], preferred_element_type=jnp.float32)
                    + b1[...])
    h = jax.nn.relu(jnp.dot(h, w2[...], preferred_element_type=jnp.float32)
                    + b2[...])
    h = jnp.dot(h, w3[...], preferred_element_type=jnp.float32) + b3[...]
    nfv = nf[...]
    z = jnp.dot(nfv, ws[...], preferred_element_type=jnp.float32) + bs[...]
    z += jnp.dot(h, wn[...], preferred_element_type=jnp.float32) + bn[...]
    o[...] = jax.nn.sigmoid(z) + nfv


def _tc_node(m0a, m1a, m0b, m1b, nf, b0, w1, b1, w2, b2, w3, b3,
             ws, bs, wn, bn):
    full = lambda r, c: pl.BlockSpec((r, c), lambda i: (0, 0))
    mspec = pl.BlockSpec((_BN, HD0), lambda i: (i, 0))
    return pl.pallas_call(
        _tc_node_body,
        grid=(N // _BN,),
        in_specs=[
            mspec, mspec, mspec, mspec,
            pl.BlockSpec((_BN, ND), lambda i: (i, 0)),
            full(1, HD0),
            full(HD0, 64), full(1, 64),
            full(64, 128), full(1, 128),
            full(128, ND), full(1, ND),
            full(ND, ND), full(1, ND),
            full(ND, ND), full(1, ND),
        ],
        out_specs=pl.BlockSpec((_BN, ND), lambda i: (i, 0)),
        out_shape=jax.ShapeDtypeStruct((N, ND), jnp.float32),
    )(m0a, m1a, m0b, m1b, nf, b0, w1, b1, w2, b2, w3, b3, ws, bs, wn, bn)


def _tc_edge_body(s, t, er, ea, ewa, ewb, ewc, eb, wa, wb, wc, b,
                  v0, c0, v1, c1, v2, c2, v3, c3, o):
    a1 = jnp.dot(s[...], ewa[...], preferred_element_type=jnp.float32)
    a1 += jnp.dot(er[...], ewb[...], preferred_element_type=jnp.float32)
    a1 += jnp.dot(ea[...], ewc[...], preferred_element_type=jnp.float32)
    a2 = jnp.dot(t[...], wa[...], preferred_element_type=jnp.float32)
    a2 += jnp.dot(er[...], wb[...], preferred_element_type=jnp.float32)
    a2 += jnp.dot(ea[...], wc[...], preferred_element_type=jnp.float32)
    g = jax.nn.sigmoid(a1 + eb[...]) + jax.nn.sigmoid(a2 + b[...])
    g = jax.nn.relu(jnp.dot(g, v0[...], preferred_element_type=jnp.float32)
                    + c0[...])
    g = jax.nn.relu(jnp.dot(g, v1[...], preferred_element_type=jnp.float32)
                    + c1[...])
    g = jax.nn.relu(jnp.dot(g, v2[...], preferred_element_type=jnp.float32)
                    + c2[...])
    o[...] = jnp.dot(g, v3[...], preferred_element_type=jnp.float32) + c3[...]


def _tc_edge(s, t, er, ea, ewa, ewb, ewc, eb, wa, wb, wc, b,
             v0, c0, v1, c1, v2, c2, v3, c3):
    total = s.shape[0]
    bf = total // 32
    full = lambda r, c: pl.BlockSpec((r, c), lambda i: (0, 0))
    return pl.pallas_call(
        _tc_edge_body,
        grid=(32,),
        in_specs=[
            pl.BlockSpec((bf, ND), lambda i: (i, 0)),
            pl.BlockSpec((bf, ND), lambda i: (i, 0)),
            pl.BlockSpec((bf, ERD), lambda i: (i, 0)),
            pl.BlockSpec((bf, EAD), lambda i: (i, 0)),
            full(ND, ACD), full(ERD, ACD), full(EAD, ACD), full(1, ACD),
            full(ND, ACD), full(ERD, ACD), full(EAD, ACD), full(1, ACD),
            full(ACD, 148), full(1, 148),
            full(148, 8), full(1, 8),
            full(8, 16), full(1, 16),
            full(16, 32), full(1, 32),
        ],
        out_specs=pl.BlockSpec((bf, 32), lambda i: (i, 0)),
        out_shape=jax.ShapeDtypeStruct((total, 32), jnp.float32),
    )(s, t, er, ea, ewa, ewb, ewc, eb, wa, wb, wc, b,
      v0, c0, v1, c1, v2, c2, v3, c3)


def kernel(node_features, edge_radial, edge_angular, edge_index,
           mlp1_w, mlp1_b, mlp2_w0, mlp2_b0, mlp2_w1, mlp2_b1,
           mlp2_w2, mlp2_b2, mlp2_w3, mlp2_b3, self_w, self_b,
           neigh_w, neigh_b, emlp1_w, emlp1_b, emlp2_w0, emlp2_b0,
           emlp2_w1, emlp2_b1, emlp2_w2, emlp2_b2, emlp2_w3, emlp2_b3):
    src = edge_index[0]
    dst = edge_index[1]

    w1t = mlp1_w.T
    wa, wb, wc = w1t[:ND], w1t[ND:ND + ERD], w1t[ND + ERD:]
    b1r = mlp1_b[None, :]
    e1t = emlp1_w.T
    ewa, ewb, ewc = e1t[:ND], e1t[ND:ND + ERD], e1t[ND + ERD:]
    eb1r = emlp1_b[None, :]
    w0p = mlp2_w0.T
    zeros = jnp.zeros((_NRPS, HD0), jnp.float32)

    def _split(x):
        xw = x.reshape(NW, _EPW, *x.shape[1:])
        return (xw[:, :_H_EPW[0]].reshape(-1, *x.shape[1:]),
                xw[:, _H_EPW[0]:].reshape(-1, *x.shape[1:]))

    src_h = _split(src)
    dst_h = _split(dst)
    er_h = _split(edge_radial)
    ea_h = _split(edge_angular)
    src2d_g = tuple(src_h[h].reshape(-1, _GCH) for h in range(2))
    dst2d_g = tuple(dst_h[h].reshape(-1, _GCH) for h in range(2))
    dst2d_c = tuple(dst_h[h].reshape(-1, _H_CH[h]) for h in range(2))

    z = [None, None]
    for h in range(2):
        x2 = _sc_a[h](node_features, src2d_g[h], src)
        z[h] = _tc_sub(x2, er_h[h], ea_h[h], wa, wb, wc, b1r, w0p)

    m0a, m1a = _sc_c[0](z[0], dst2d_c[0], zeros)
    m0b, m1b = _sc_c[1](z[1], dst2d_c[1], zeros)

    nfu = _tc_node(m0a[:N], m1a[:N], m0b[:N], m1b[:N], node_features,
                   mlp2_b0[None, :],
                   mlp2_w1.T, mlp2_b1[None, :],
                   mlp2_w2.T, mlp2_b2[None, :],
                   mlp2_w3.T, mlp2_b3[None, :],
                   self_w.T, self_b[None, :],
                   neigh_w.T, neigh_b[None, :])

    e_h = [None, None]
    for h in range(2):
        s_rows, t_rows = _sc_e[h](nfu, src2d_g[h], dst2d_g[h])
        e_h[h] = _tc_edge(s_rows, t_rows, er_h[h], ea_h[h],
                          ewa, ewb, ewc, eb1r, wa, wb, wc, b1r,
                          emlp2_w0.T, emlp2_b0[None, :],
                          emlp2_w1.T, emlp2_b1[None, :],
                          emlp2_w2.T, emlp2_b2[None, :],
                          emlp2_w3.T, emlp2_b3[None, :])

    e = jnp.concatenate([e_h[0].reshape(NW, _H_EPW[0], 32),
                         e_h[1].reshape(NW, _H_EPW[1], 32)],
                        axis=1).reshape(E, 32)
    return (nfu, e)


# speedup vs baseline: 1.9417x; 1.1473x over previous
import functools

import jax
import jax.numpy as jnp
from jax import lax
from jax.experimental import pallas as pl
from jax.experimental.pallas import tpu as pltpu
from jax.experimental.pallas import tpu_sc as plsc

N = 10000
E = 160000
ND = 256
ERD = 16
EAD = 16
ACD = ND + ERD + EAD
HD0 = 176
ZP = 256
HP = 128

NC = 2
NS = 16
NW = NC * NS

_POFF = (0, 51200, 102400)
_PLEN = (51200, 51200, 57600)
_CH = 200
_TB = 800

_NRPS = 640
_NPAD = NS * _NRPS

_vmesh = plsc.VectorSubcoreMesh(core_axis_name="c", subcore_axis_name="s")


def _mk_gather_compose(off, ln):
    epw = ln // NW
    nch = epw // _CH

    @functools.partial(
        pl.kernel,
        mesh=_vmesh,
        out_type=jax.ShapeDtypeStruct((ln, ND), jnp.float32),
        scratch_types=[
            pltpu.VMEM((1, _CH), jnp.int32),
            pltpu.VMEM((1, _CH), jnp.int32),
            pltpu.VMEM((_CH, ND), jnp.float32),
        ],
    )
    def _gather_compose(nf_hbm, src2d_hbm, srcfull_hbm, x2_hbm,
                        srcv, src2v, rows):
        wid = lax.axis_index("s") * NC + lax.axis_index("c")

        @pl.loop(0, nch)
        def _(k):
            crow = (off + wid * epw) // _CH + k
            pltpu.sync_copy(src2d_hbm.at[pl.ds(crow, 1)], srcv)
            pltpu.sync_copy(srcfull_hbm.at[srcv.at[0]], src2v.at[0])
            pltpu.sync_copy(nf_hbm.at[src2v.at[0]], rows)
            pltpu.sync_copy(rows, x2_hbm.at[pl.ds(wid * epw + k * _CH, _CH)])

    return _gather_compose


def _mk_scatter_add(off, ln):
    eps = ln // NS
    nch = eps // _CH

    @functools.partial(
        pl.kernel,
        mesh=_vmesh,
        out_type=[
            jax.ShapeDtypeStruct((_NPAD, HP), jnp.float32),
            jax.ShapeDtypeStruct((_NPAD, HP), jnp.float32),
        ],
        scratch_types=[
            pltpu.VMEM((1, _CH), jnp.int32),
            pltpu.VMEM((_CH, HP), jnp.float32),
            pltpu.VMEM_SHARED((_NPAD, HP), jnp.float32),
        ],
    )
    def _scatter_add(z_hbm, dst2d_hbm, zeros_hbm, m0_hbm, m1_hbm,
                     dstv, rows, acc):
        cid = lax.axis_index("c")
        sid = lax.axis_index("s")

        pltpu.sync_copy(zeros_hbm, acc.at[pl.ds(sid * _NRPS, _NRPS)])
        plsc.subcore_barrier()

        @pl.loop(0, nch)
        def _(k):
            base = sid * eps + k * _CH
            pltpu.sync_copy(dst2d_hbm.at[pl.ds((off + base) // _CH, 1)], dstv)

            @pl.when(cid == 0)
            def _():
                pltpu.sync_copy(z_hbm.at[0, pl.ds(base, _CH)], rows)

            @pl.when(cid == 1)
            def _():
                pltpu.sync_copy(z_hbm.at[1, pl.ds(base, _CH)], rows)

            pltpu.sync_copy(rows, acc.at[dstv.at[0]], add=True)

        plsc.subcore_barrier()
        out_rows = pl.ds(sid * _NRPS, _NRPS)

        @pl.when(cid == 0)
        def _():
            pltpu.sync_copy(acc.at[out_rows], m0_hbm.at[out_rows])

        @pl.when(cid == 1)
        def _():
            pltpu.sync_copy(acc.at[out_rows], m1_hbm.at[out_rows])

    return _scatter_add


def _mk_gather_pair(off, ln):
    epw = ln // NW
    nch = epw // _CH

    @functools.partial(
        pl.kernel,
        mesh=_vmesh,
        out_type=[
            jax.ShapeDtypeStruct((ln, ND), jnp.float32),
            jax.ShapeDtypeStruct((ln, ND), jnp.float32),
        ],
        scratch_types=[
            pltpu.VMEM((1, _CH), jnp.int32),
            pltpu.VMEM((1, _CH), jnp.int32),
            pltpu.VMEM((_CH, ND), jnp.float32),
            pltpu.VMEM((_CH, ND), jnp.float32),
        ],
    )
    def _gather_pair(nfu_hbm, src2d_hbm, dst2d_hbm, s_hbm, t_hbm,
                     srcv, dstv, rs, rt):
        wid = lax.axis_index("s") * NC + lax.axis_index("c")

        @pl.loop(0, nch)
        def _(k):
            crow = pl.ds((off + wid * epw) // _CH + k, 1)
            pltpu.sync_copy(src2d_hbm.at[crow], srcv)
            pltpu.sync_copy(dst2d_hbm.at[crow], dstv)
            pltpu.sync_copy(nfu_hbm.at[srcv.at[0]], rs)
            pltpu.sync_copy(nfu_hbm.at[dstv.at[0]], rt)
            sl = pl.ds(wid * epw + k * _CH, _CH)
            pltpu.sync_copy(rs, s_hbm.at[sl])
            pltpu.sync_copy(rt, t_hbm.at[sl])

    return _gather_pair


_sc_a = tuple(_mk_gather_compose(_POFF[p], _PLEN[p]) for p in range(3))
_sc_c = tuple(_mk_scatter_add(_POFF[p], _PLEN[p]) for p in range(3))
_sc_e = tuple(_mk_gather_pair(_POFF[p], _PLEN[p]) for p in range(3))


def _tc_sub_body(x2, er, ea, wa, wb, wc, b, w0p, o):
    acc = jnp.dot(x2[...], wa[...], preferred_element_type=jnp.float32)
    acc += jnp.dot(er[...], wb[...], preferred_element_type=jnp.float32)
    acc += jnp.dot(ea[...], wc[...], preferred_element_type=jnp.float32)
    sub = jax.nn.sigmoid(acc + b[...])
    z = jnp.dot(sub, w0p[...], preferred_element_type=jnp.float32)
    o[0] = z[:, :HP]
    o[1] = z[:, HP:]


def _tc_sub(p, x2, er, ea, wa, wb, wc, b, w0p):
    ln = _PLEN[p]
    ob = _POFF[p] // _TB
    full = lambda r, c: pl.BlockSpec((r, c), lambda i: (0, 0))
    return pl.pallas_call(
        _tc_sub_body,
        grid=(ln // _TB,),
        in_specs=[
            pl.BlockSpec((_TB, ND), lambda i: (i, 0)),
            pl.BlockSpec((_TB, ERD), lambda i: (ob + i, 0)),
            pl.BlockSpec((_TB, EAD), lambda i: (ob + i, 0)),
            full(ND, ACD), full(ERD, ACD), full(EAD, ACD), full(1, ACD),
            full(ACD, ZP),
        ],
        out_specs=pl.BlockSpec((2, _TB, HP), lambda i: (0, i, 0)),
        out_shape=jax.ShapeDtypeStruct((2, ln, HP), jnp.float32),
    )(x2, er, ea, wa, wb, wc, b, w0p)


_BN = 1000


def _tc_node_body(m0a, m0b, m0c, m1a, m1b, m1c, nf, s0, s1,
                  b0, w1, b1, w2, b2, w3, b3, ws, bs, wn, bn, o):
    p0 = m0a[...] + m0b[...] + m0c[...]
    p1 = m1a[...] + m1b[...] + m1c[...]
    hpre = jnp.dot(p0, s0[...], preferred_element_type=jnp.float32)
    hpre += jnp.dot(p1, s1[...], preferred_element_type=jnp.float32)
    h = jax.nn.relu(hpre + b0[...])
    h = jax.nn.relu(jnp.dot(h, w1[...], preferred_element_type=jnp.float32)
                    + b1[...])
    h = jax.nn.relu(jnp.dot(h, w2[...], preferred_element_type=jnp.float32)
                    + b2[...])
    h = jnp.dot(h, w3[...], preferred_element_type=jnp.float32) + b3[...]
    nfv = nf[...]
    z = jnp.dot(nfv, ws[...], preferred_element_type=jnp.float32) + bs[...]
    z += jnp.dot(h, wn[...], preferred_element_type=jnp.float32) + bn[...]
    o[...] = jax.nn.sigmoid(z) + nfv


def _tc_node(ms, nf, s0, s1, b0, w1, b1, w2, b2, w3, b3, ws, bs, wn, bn):
    full = lambda r, c: pl.BlockSpec((r, c), lambda i: (0, 0))
    mspec = pl.BlockSpec((_BN, HP), lambda i: (i, 0))
    return pl.pallas_call(
        _tc_node_body,
        grid=(N // _BN,),
        in_specs=[
            mspec, mspec, mspec, mspec, mspec, mspec,
            pl.BlockSpec((_BN, ND), lambda i: (i, 0)),
            full(HP, HD0), full(HP, HD0),
            full(1, HD0),
            full(HD0, 64), full(1, 64),
            full(64, 128), full(1, 128),
            full(128, ND), full(1, ND),
            full(ND, ND), full(1, ND),
            full(ND, ND), full(1, ND),
        ],
        out_specs=pl.BlockSpec((_BN, ND), lambda i: (i, 0)),
        out_shape=jax.ShapeDtypeStruct((N, ND), jnp.float32),
    )(*ms, nf, s0, s1, b0, w1, b1, w2, b2, w3, b3, ws, bs, wn, bn)


def _tc_edge_body(s, t, er, ea, ewa, ewb, ewc, eb, wa, wb, wc, b,
                  v0, c0, v1, c1, v2, c2, v3, c3, o):
    a1 = jnp.dot(s[...], ewa[...], preferred_element_type=jnp.float32)
    a1 += jnp.dot(er[...], ewb[...], preferred_element_type=jnp.float32)
    a1 += jnp.dot(ea[...], ewc[...], preferred_element_type=jnp.float32)
    a2 = jnp.dot(t[...], wa[...], preferred_element_type=jnp.float32)
    a2 += jnp.dot(er[...], wb[...], preferred_element_type=jnp.float32)
    a2 += jnp.dot(ea[...], wc[...], preferred_element_type=jnp.float32)
    g = jax.nn.sigmoid(a1 + eb[...]) + jax.nn.sigmoid(a2 + b[...])
    g = jax.nn.relu(jnp.dot(g, v0[...], preferred_element_type=jnp.float32)
                    + c0[...])
    g = jax.nn.relu(jnp.dot(g, v1[...], preferred_element_type=jnp.float32)
                    + c1[...])
    g = jax.nn.relu(jnp.dot(g, v2[...], preferred_element_type=jnp.float32)
                    + c2[...])
    o[...] = jnp.dot(g, v3[...], preferred_element_type=jnp.float32) + c3[...]


def _tc_edge(p, s, t, er, ea, ewa, ewb, ewc, eb, wa, wb, wc, b,
             v0, c0, v1, c1, v2, c2, v3, c3):
    ln = _PLEN[p]
    ob = _POFF[p] // _TB
    full = lambda r, c: pl.BlockSpec((r, c), lambda i: (0, 0))
    return pl.pallas_call(
        _tc_edge_body,
        grid=(ln // _TB,),
        in_specs=[
            pl.BlockSpec((_TB, ND), lambda i: (i, 0)),
            pl.BlockSpec((_TB, ND), lambda i: (i, 0)),
            pl.BlockSpec((_TB, ERD), lambda i: (ob + i, 0)),
            pl.BlockSpec((_TB, EAD), lambda i: (ob + i, 0)),
            full(ND, ACD), full(ERD, ACD), full(EAD, ACD), full(1, ACD),
            full(ND, ACD), full(ERD, ACD), full(EAD, ACD), full(1, ACD),
            full(ACD, 148), full(1, 148),
            full(148, 8), full(1, 8),
            full(8, 16), full(1, 16),
            full(16, 32), full(1, 32),
        ],
        out_specs=pl.BlockSpec((_TB, 32), lambda i: (i, 0)),
        out_shape=jax.ShapeDtypeStruct((ln, 32), jnp.float32),
    )(s, t, er, ea, ewa, ewb, ewc, eb, wa, wb, wc, b,
      v0, c0, v1, c1, v2, c2, v3, c3)


def kernel(node_features, edge_radial, edge_angular, edge_index,
           mlp1_w, mlp1_b, mlp2_w0, mlp2_b0, mlp2_w1, mlp2_b1,
           mlp2_w2, mlp2_b2, mlp2_w3, mlp2_b3, self_w, self_b,
           neigh_w, neigh_b, emlp1_w, emlp1_b, emlp2_w0, emlp2_b0,
           emlp2_w1, emlp2_b1, emlp2_w2, emlp2_b2, emlp2_w3, emlp2_b3):
    src = edge_index[0]
    dst = edge_index[1]
    src2d = src.reshape(E // _CH, _CH)
    dst2d = dst.reshape(E // _CH, _CH)

    w1t = mlp1_w.T
    wa, wb, wc = w1t[:ND], w1t[ND:ND + ERD], w1t[ND + ERD:]
    b1r = mlp1_b[None, :]
    e1t = emlp1_w.T
    ewa, ewb, ewc = e1t[:ND], e1t[ND:ND + ERD], e1t[ND + ERD:]
    eb1r = emlp1_b[None, :]
    w0p = jnp.pad(mlp2_w0.T, ((0, 0), (0, ZP - HD0)))
    zeros = jnp.zeros((_NRPS, HP), jnp.float32)
    sel0 = jnp.eye(HP, HD0, dtype=jnp.float32)
    sel1 = jnp.eye(HP, HD0, k=HP, dtype=jnp.float32)

    ms = [None] * 6
    for p in range(3):
        x2 = _sc_a[p](node_features, src2d, src)
        z = _tc_sub(p, x2, edge_radial, edge_angular, wa, wb, wc, b1r, w0p)
        ms[p], ms[3 + p] = _sc_c[p](z, dst2d, zeros)

    nfu = _tc_node(ms, node_features, sel0, sel1,
                   mlp2_b0[None, :],
                   mlp2_w1.T, mlp2_b1[None, :],
                   mlp2_w2.T, mlp2_b2[None, :],
                   mlp2_w3.T, mlp2_b3[None, :],
                   self_w.T, self_b[None, :],
                   neigh_w.T, neigh_b[None, :])

    e_p = [None] * 3
    for p in range(3):
        s_rows, t_rows = _sc_e[p](nfu, src2d, dst2d)
        e_p[p] = _tc_edge(p, s_rows, t_rows, edge_radial, edge_angular,
                          ewa, ewb, ewc, eb1r, wa, wb, wc, b1r,
                          emlp2_w0.T, emlp2_b0[None, :],
                          emlp2_w1.T, emlp2_b1[None, :],
                          emlp2_w2.T, emlp2_b2[None, :],
                          emlp2_w3.T, emlp2_b3[None, :])

    return (nfu, jnp.concatenate(e_p, axis=0))


# speedup vs baseline: 2.1605x; 1.1127x over previous
import functools

import jax
import jax.numpy as jnp
from jax import lax
from jax.experimental import pallas as pl
from jax.experimental.pallas import tpu as pltpu
from jax.experimental.pallas import tpu_sc as plsc

N = 10000
E = 160000
ND = 256
ERD = 16
EAD = 16
ACD = ND + ERD + EAD
HD0 = 176
ZP = 256
HP = 128

NC = 2
NS = 16
NW = NC * NS

_POFF = (0, 51200, 102400)
_PLEN = (51200, 51200, 57600)
_CH = 200
_TB = 3200

_NRPS = 640
_NPAD = NS * _NRPS

_vmesh = plsc.VectorSubcoreMesh(core_axis_name="c", subcore_axis_name="s")


def _mk_gather_compose(off, ln):
    epw = ln // NW
    nch = epw // _CH

    @functools.partial(
        pl.kernel,
        mesh=_vmesh,
        out_type=jax.ShapeDtypeStruct((ln, ND), jnp.float32),
        scratch_types=[
            pltpu.VMEM((1, _CH), jnp.int32),
            pltpu.VMEM((1, _CH), jnp.int32),
            pltpu.VMEM((_CH, ND), jnp.float32),
        ],
    )
    def _gather_compose(nf_hbm, src2d_hbm, srcfull_hbm, x2_hbm,
                        srcv, src2v, rows):
        wid = lax.axis_index("s") * NC + lax.axis_index("c")

        @pl.loop(0, nch)
        def _(k):
            crow = (off + wid * epw) // _CH + k
            pltpu.sync_copy(src2d_hbm.at[pl.ds(crow, 1)], srcv)
            pltpu.sync_copy(srcfull_hbm.at[srcv.at[0]], src2v.at[0])
            pltpu.sync_copy(nf_hbm.at[src2v.at[0]], rows)
            pltpu.sync_copy(rows, x2_hbm.at[pl.ds(wid * epw + k * _CH, _CH)])

    return _gather_compose


def _mk_scatter_add(off, ln):
    eps = ln // NS
    nch = eps // _CH

    @functools.partial(
        pl.kernel,
        mesh=_vmesh,
        out_type=[
            jax.ShapeDtypeStruct((_NPAD, HP), jnp.float32),
            jax.ShapeDtypeStruct((_NPAD, HP), jnp.float32),
        ],
        scratch_types=[
            pltpu.VMEM((1, _CH), jnp.int32),
            pltpu.VMEM((_CH, HP), jnp.float32),
            pltpu.VMEM_SHARED((_NPAD, HP), jnp.float32),
        ],
    )
    def _scatter_add(z_hbm, dst2d_hbm, zeros_hbm, m0_hbm, m1_hbm,
                     dstv, rows, acc):
        cid = lax.axis_index("c")
        sid = lax.axis_index("s")

        pltpu.sync_copy(zeros_hbm, acc.at[pl.ds(sid * _NRPS, _NRPS)])
        plsc.subcore_barrier()

        @pl.loop(0, nch)
        def _(k):
            base = sid * eps + k * _CH
            pltpu.sync_copy(dst2d_hbm.at[pl.ds((off + base) // _CH, 1)], dstv)

            @pl.when(cid == 0)
            def _():
                pltpu.sync_copy(z_hbm.at[0, pl.ds(base, _CH)], rows)

            @pl.when(cid == 1)
            def _():
                pltpu.sync_copy(z_hbm.at[1, pl.ds(base, _CH)], rows)

            pltpu.sync_copy(rows, acc.at[dstv.at[0]], add=True)

        plsc.subcore_barrier()
        out_rows = pl.ds(sid * _NRPS, _NRPS)

        @pl.when(cid == 0)
        def _():
            pltpu.sync_copy(acc.at[out_rows], m0_hbm.at[out_rows])

        @pl.when(cid == 1)
        def _():
            pltpu.sync_copy(acc.at[out_rows], m1_hbm.at[out_rows])

    return _scatter_add


def _mk_gather_pair(off, ln):
    epw = ln // NW
    nch = epw // _CH

    @functools.partial(
        pl.kernel,
        mesh=_vmesh,
        out_type=[
            jax.ShapeDtypeStruct((ln, ND), jnp.float32),
            jax.ShapeDtypeStruct((ln, ND), jnp.float32),
        ],
        scratch_types=[
            pltpu.VMEM((1, _CH), jnp.int32),
            pltpu.VMEM((1, _CH), jnp.int32),
            pltpu.VMEM((_CH, ND), jnp.float32),
            pltpu.VMEM((_CH, ND), jnp.float32),
        ],
    )
    def _gather_pair(nfu_hbm, src2d_hbm, dst2d_hbm, s_hbm, t_hbm,
                     srcv, dstv, rs, rt):
        wid = lax.axis_index("s") * NC + lax.axis_index("c")

        @pl.loop(0, nch)
        def _(k):
            crow = pl.ds((off + wid * epw) // _CH + k, 1)
            pltpu.sync_copy(src2d_hbm.at[crow], srcv)
            pltpu.sync_copy(dst2d_hbm.at[crow], dstv)
            pltpu.sync_copy(nfu_hbm.at[srcv.at[0]], rs)
            pltpu.sync_copy(nfu_hbm.at[dstv.at[0]], rt)
            sl = pl.ds(wid * epw + k * _CH, _CH)
            pltpu.sync_copy(rs, s_hbm.at[sl])
            pltpu.sync_copy(rt, t_hbm.at[sl])

    return _gather_pair


_sc_a = tuple(_mk_gather_compose(_POFF[p], _PLEN[p]) for p in range(3))
_sc_c = tuple(_mk_scatter_add(_POFF[p], _PLEN[p]) for p in range(3))
_sc_e = tuple(_mk_gather_pair(_POFF[p], _PLEN[p]) for p in range(3))


def _bdot(x, w):
    return jnp.dot(x.astype(jnp.bfloat16), w[...],
                   preferred_element_type=jnp.float32)


def _tc_sub_body(x2, er, ea, wa, wb, wc, b, w0p, o):
    acc = _bdot(x2[...], wa) + _bdot(er[...], wb) + _bdot(ea[...], wc)
    sub = jax.nn.sigmoid(acc + b[...])
    z = _bdot(sub, w0p)
    o[0] = z[:, :HP]
    o[1] = z[:, HP:]


def _tc_sub(p, x2, er, ea, wa, wb, wc, b, w0p):
    ln = _PLEN[p]
    ob = _POFF[p] // _TB
    full = lambda r, c: pl.BlockSpec((r, c), lambda i: (0, 0))
    return pl.pallas_call(
        _tc_sub_body,
        grid=(ln // _TB,),
        in_specs=[
            pl.BlockSpec((_TB, ND), lambda i: (i, 0)),
            pl.BlockSpec((_TB, ERD), lambda i: (ob + i, 0)),
            pl.BlockSpec((_TB, EAD), lambda i: (ob + i, 0)),
            full(ND, ACD), full(ERD, ACD), full(EAD, ACD), full(1, ACD),
            full(ACD, ZP),
        ],
        out_specs=pl.BlockSpec((2, _TB, HP), lambda i: (0, i, 0)),
        out_shape=jax.ShapeDtypeStruct((2, ln, HP), jnp.float32),
    )(x2, er, ea, wa, wb, wc, b, w0p)


_BN = 1000


def _tc_node_body(m0a, m0b, m0c, m1a, m1b, m1c, nf, s0, s1,
                  b0, w1, b1, w2, b2, w3, b3, ws, bs, wn, bn, o):
    p0 = m0a[...] + m0b[...] + m0c[...]
    p1 = m1a[...] + m1b[...] + m1c[...]
    hpre = jnp.dot(p0, s0[...], preferred_element_type=jnp.float32)
    hpre += jnp.dot(p1, s1[...], preferred_element_type=jnp.float32)
    h = jax.nn.relu(hpre + b0[...])
    h = jax.nn.relu(jnp.dot(h, w1[...], preferred_element_type=jnp.float32)
                    + b1[...])
    h = jax.nn.relu(jnp.dot(h, w2[...], preferred_element_type=jnp.float32)
                    + b2[...])
    h = jnp.dot(h, w3[...], preferred_element_type=jnp.float32) + b3[...]
    nfv = nf[...]
    z = jnp.dot(nfv, ws[...], preferred_element_type=jnp.float32) + bs[...]
    z += jnp.dot(h, wn[...], preferred_element_type=jnp.float32) + bn[...]
    o[...] = jax.nn.sigmoid(z) + nfv


def _tc_node(ms, nf, s0, s1, b0, w1, b1, w2, b2, w3, b3, ws, bs, wn, bn):
    full = lambda r, c: pl.BlockSpec((r, c), lambda i: (0, 0))
    mspec = pl.BlockSpec((_BN, HP), lambda i: (i, 0))
    return pl.pallas_call(
        _tc_node_body,
        grid=(N // _BN,),
        in_specs=[
            mspec, mspec, mspec, mspec, mspec, mspec,
            pl.BlockSpec((_BN, ND), lambda i: (i, 0)),
            full(HP, HD0), full(HP, HD0),
            full(1, HD0),
            full(HD0, 64), full(1, 64),
            full(64, 128), full(1, 128),
            full(128, ND), full(1, ND),
            full(ND, ND), full(1, ND),
            full(ND, ND), full(1, ND),
        ],
        out_specs=pl.BlockSpec((_BN, ND), lambda i: (i, 0)),
        out_shape=jax.ShapeDtypeStruct((N, ND), jnp.float32),
    )(*ms, nf, s0, s1, b0, w1, b1, w2, b2, w3, b3, ws, bs, wn, bn)


def _tc_edge_body(s, t, er, ea, ewa, ewb, ewc, eb, wa, wb, wc, b,
                  v0, c0, v1, c1, v2, c2, v3, c3, o):
    a1 = _bdot(s[...], ewa) + _bdot(er[...], ewb) + _bdot(ea[...], ewc)
    a2 = _bdot(t[...], wa) + _bdot(er[...], wb) + _bdot(ea[...], wc)
    g = jax.nn.sigmoid(a1 + eb[...]) + jax.nn.sigmoid(a2 + b[...])
    g = jax.nn.relu(_bdot(g, v0) + c0[...])
    g = jax.nn.relu(_bdot(g, v1) + c1[...])
    g = jax.nn.relu(_bdot(g, v2) + c2[...])
    o[...] = _bdot(g, v3) + c3[...]


def _tc_edge(p, s, t, er, ea, ewa, ewb, ewc, eb, wa, wb, wc, b,
             v0, c0, v1, c1, v2, c2, v3, c3):
    ln = _PLEN[p]
    ob = _POFF[p] // _TB
    full = lambda r, c: pl.BlockSpec((r, c), lambda i: (0, 0))
    return pl.pallas_call(
        _tc_edge_body,
        grid=(ln // _TB,),
        in_specs=[
            pl.BlockSpec((_TB, ND), lambda i: (i, 0)),
            pl.BlockSpec((_TB, ND), lambda i: (i, 0)),
            pl.BlockSpec((_TB, ERD), lambda i: (ob + i, 0)),
            pl.BlockSpec((_TB, EAD), lambda i: (ob + i, 0)),
            full(ND, ACD), full(ERD, ACD), full(EAD, ACD), full(1, ACD),
            full(ND, ACD), full(ERD, ACD), full(EAD, ACD), full(1, ACD),
            full(ACD, 148), full(1, 148),
            full(148, 8), full(1, 8),
            full(8, 16), full(1, 16),
            full(16, 32), full(1, 32),
        ],
        out_specs=pl.BlockSpec((_TB, 32), lambda i: (i, 0)),
        out_shape=jax.ShapeDtypeStruct((ln, 32), jnp.float32),
    )(s, t, er, ea, ewa, ewb, ewc, eb, wa, wb, wc, b,
      v0, c0, v1, c1, v2, c2, v3, c3)


def kernel(node_features, edge_radial, edge_angular, edge_index,
           mlp1_w, mlp1_b, mlp2_w0, mlp2_b0, mlp2_w1, mlp2_b1,
           mlp2_w2, mlp2_b2, mlp2_w3, mlp2_b3, self_w, self_b,
           neigh_w, neigh_b, emlp1_w, emlp1_b, emlp2_w0, emlp2_b0,
           emlp2_w1, emlp2_b1, emlp2_w2, emlp2_b2, emlp2_w3, emlp2_b3):
    src = edge_index[0]
    dst = edge_index[1]
    src2d = src.reshape(E // _CH, _CH)
    dst2d = dst.reshape(E // _CH, _CH)

    bf = jnp.bfloat16
    w1t = mlp1_w.T.astype(bf)
    wa, wb, wc = w1t[:ND], w1t[ND:ND + ERD], w1t[ND + ERD:]
    b1r = mlp1_b[None, :]
    e1t = emlp1_w.T.astype(bf)
    ewa, ewb, ewc = e1t[:ND], e1t[ND:ND + ERD], e1t[ND + ERD:]
    eb1r = emlp1_b[None, :]
    w0p = jnp.pad(mlp2_w0.T, ((0, 0), (0, ZP - HD0))).astype(bf)
    zeros = jnp.zeros((_NRPS, HP), jnp.float32)
    sel0 = jnp.eye(HP, HD0, dtype=jnp.float32)
    sel1 = jnp.eye(HP, HD0, k=HP, dtype=jnp.float32)

    ms = [None] * 6
    for p in range(3):
        x2 = _sc_a[p](node_features, src2d, src)
        z = _tc_sub(p, x2, edge_radial, edge_angular, wa, wb, wc, b1r, w0p)
        ms[p], ms[3 + p] = _sc_c[p](z, dst2d, zeros)

    nfu = _tc_node(ms, node_features, sel0, sel1,
                   mlp2_b0[None, :],
                   mlp2_w1.T, mlp2_b1[None, :],
                   mlp2_w2.T, mlp2_b2[None, :],
                   mlp2_w3.T, mlp2_b3[None, :],
                   self_w.T, self_b[None, :],
                   neigh_w.T, neigh_b[None, :])

    e_p = [None] * 3
    for p in range(3):
        s_rows, t_rows = _sc_e[p](nfu, src2d, dst2d)
        e_p[p] = _tc_edge(p, s_rows, t_rows, edge_radial, edge_angular,
                          ewa, ewb, ewc, eb1r, wa, wb, wc, b1r,
                          emlp2_w0.T.astype(bf), emlp2_b0[None, :],
                          emlp2_w1.T.astype(bf), emlp2_b1[None, :],
                          emlp2_w2.T.astype(bf), emlp2_b2[None, :],
                          emlp2_w3.T.astype(bf), emlp2_b3[None, :])

    return (nfu, jnp.concatenate(e_p, axis=0))
